# Initial kernel scaffold; baseline (speedup 1.0000x reference)
import jax
import jax.numpy as jnp
from jax.experimental import pallas as pl


def kernel(x, edge_index, adj_values, W, alpha):
    raise NotImplementedError("write your pallas kernel here")


# speedup vs baseline: 3.4341x; 3.4341x over previous
import functools

import jax
import jax.numpy as jnp
from jax import lax
from jax.experimental import pallas as pl
from jax.experimental.pallas import tpu as pltpu
from jax.experimental.pallas import tpu_sc as plsc

N_NODES = 10000
N_EDGES = 160000
D_IN = 256
D_OUT = 256

NC = 2
NS = 16
L = 16

DH = D_OUT // 2
CHUNK = 128
N_CHUNKS = N_EDGES // CHUNK
N_PAD = 10240
ROWS_PER_TILE = N_PAD // NS
DRAIN = 128
N_DRAIN = ROWS_PER_TILE // DRAIN


def _matmul_kernel(x_ref, w_ref, out_ref):
    out_ref[0] = jnp.dot(x_ref[...], w_ref[...],
                         preferred_element_type=jnp.float32)


def _matmul_split(x, w):
    m_blk = 1000
    grid = (NC, N_NODES // m_blk)
    return pl.pallas_call(
        _matmul_kernel,
        grid=grid,
        in_specs=[
            pl.BlockSpec((m_blk, D_IN), lambda i, j: (j, 0)),
            pl.BlockSpec((D_IN, DH), lambda i, j: (0, i)),
        ],
        out_specs=pl.BlockSpec((1, m_blk, DH), lambda i, j: (i, j, 0)),
        out_shape=jax.ShapeDtypeStruct((NC, N_NODES, DH), jnp.float32),
    )(x, w)


def _spmm_body(h_hbm, packed_hbm, alpha_hbm, out_hbm,
               acc, eb, colb, gbuf, alphab, sem):
    c = lax.axis_index("c")
    s = lax.axis_index("s")

    zero = jnp.zeros((L,), jnp.float32)

    def zrow(r, _):
        for j in range(DH // L):
            gbuf[r, pl.ds(j * L, L)] = zero
        return 0

    lax.fori_loop(0, CHUNK, zrow, 0)

    r0 = s * ROWS_PER_TILE
    for d in range(N_DRAIN):
        pltpu.sync_copy(gbuf.at[pl.ds(0, DRAIN)],
                        acc.at[pl.ds(r0 + d * DRAIN, DRAIN)])

    pltpu.sync_copy(alpha_hbm.at[pl.ds(c * DH, DH)], alphab)

    plsc.subcore_barrier()

    col_off = (c * N_NODES).astype(jnp.int32)
    base_chunks = N_CHUNKS // NS
    rem_chunks = N_CHUNKS % NS
    n_my_chunks = base_chunks + jnp.where(s < rem_chunks, 1, 0)

    def chunk_body(i, _):
        chunk_id = s + i * NS
        pltpu.sync_copy(packed_hbm.at[chunk_id], eb)
        for j in range(CHUNK // L):
            colb[pl.ds(j * L, L)] = eb[1, pl.ds(j * L, L)] + col_off
        pltpu.async_copy(h_hbm.at[colb], gbuf, sem).wait()
        def edge_body(e, _):
            idx = jnp.zeros((L,), jnp.int32) + e
            w_bits = plsc.load_gather(eb.at[2], [idx])
            w = plsc.bitcast(w_bits, jnp.float32)
            for j in range(DH // L):
                sl = pl.ds(j * L, L)
                gbuf[e, sl] = gbuf[e, sl] * w
            return 0

        lax.fori_loop(0, CHUNK, edge_body, 0)
        pltpu.sync_copy(gbuf, acc.at[eb.at[0]], add=True)
        return 0

    lax.fori_loop(0, n_my_chunks, chunk_body, 0)

    plsc.subcore_barrier()

    for d in range(N_DRAIN):
        rbase = r0 + d * DRAIN
        pltpu.sync_copy(acc.at[pl.ds(rbase, DRAIN)], gbuf.at[pl.ds(0, DRAIN)])

        def prow(r, _):
            for j in range(DH // L):
                sl = pl.ds(j * L, L)
                v = gbuf[r, sl]
                a = alphab[sl]
                gbuf[r, sl] = jnp.where(v >= 0.0, v, a * v)
            return 0

        lax.fori_loop(0, DRAIN, prow, 0)
        pltpu.sync_copy(gbuf.at[pl.ds(0, DRAIN)],
                        out_hbm.at[pl.ds(rbase, DRAIN), pl.ds(c * DH, DH)])


_spmm = functools.partial(
    pl.kernel,
    out_type=jax.ShapeDtypeStruct((N_PAD, D_OUT), jnp.float32),
    mesh=plsc.VectorSubcoreMesh(core_axis_name="c", subcore_axis_name="s",
                                num_cores=NC, num_subcores=NS),
    scratch_types=[
        pltpu.MemorySpace.VMEM_SHARED((N_PAD, DH), jnp.float32),
        pltpu.VMEM((3, CHUNK), jnp.int32),
        pltpu.VMEM((CHUNK,), jnp.int32),
        pltpu.VMEM((CHUNK, DH), jnp.float32),
        pltpu.VMEM((DH,), jnp.float32),
        pltpu.SemaphoreType.DMA,
    ],
    compiler_params=pltpu.CompilerParams(needs_layout_passes=False),
)(_spmm_body)


def kernel(x, edge_index, adj_values, W, alpha):
    ei = edge_index.astype(jnp.int32)
    packed = jnp.stack(
        [ei[0], ei[1], lax.bitcast_convert_type(adj_values, jnp.int32)])
    packed = packed.reshape(3, N_CHUNKS, CHUNK).transpose(1, 0, 2)
    h = _matmul_split(x, W)
    h_flat = h.reshape(NC * N_NODES, DH)
    out = _spmm(h_flat, packed, alpha)
    return out[:N_NODES]
